# baseline (speedup 1.0000x reference)
import functools
import jax
import jax.numpy as jnp
from jax import lax
from jax.experimental import pallas as pl
from jax.experimental.pallas import tpu as pltpu, tpu_sc as plsc

_BATCH = 16384
_DIM = 256
_NC = 2
_NS = 16
_NW = _NC * _NS
_ROWS_W = _BATCH // _NW
_CHUNK = 64
_NCHUNK = _ROWS_W // _CHUNK
_RING = 3
_PF = 2


def _dot_chunk(gu_v, gi_v, xui_v):
    lane = lax.iota(jnp.int32, 16)
    for g in range(_CHUNK // 16):
        row_idx = g * 16 + lane

        def col_body(c, acc):
            colv = (lane * 8 + c) & (_DIM - 1)
            return acc + (
                plsc.load_gather(gu_v, [row_idx, colv])
                * plsc.load_gather(gi_v, [row_idx, colv])
            )

        acc = lax.fori_loop(0, _DIM, col_body, jnp.zeros((16,), jnp.float32), unroll=8)
        xui_v[pl.ds(g * 16, 16)] = acc


def _sc_body(gu_hbm, gi_hbm, xui_hbm, guo_hbm, gio_hbm, *scratch):
    gu_bufs = scratch[0:_RING]
    gi_bufs = scratch[_RING:2 * _RING]
    x_bufs = scratch[2 * _RING:3 * _RING]
    in_sems = scratch[3 * _RING:4 * _RING]
    out_sems = scratch[4 * _RING:5 * _RING]

    wid = lax.axis_index("c") * _NS + lax.axis_index("s")
    row0 = wid * _ROWS_W

    def start_in(ci, b):
        rbase = row0 + ci * _CHUNK
        h_gu = pltpu.async_copy(gu_hbm.at[pl.ds(rbase, _CHUNK)], gu_bufs[b], in_sems[b])
        h_gi = pltpu.async_copy(gi_hbm.at[pl.ds(rbase, _CHUNK)], gi_bufs[b], in_sems[b])
        return (h_gu, h_gi)

    def start_out(ci, b):
        rbase = row0 + ci * _CHUNK
        h_gu = pltpu.async_copy(gu_bufs[b], guo_hbm.at[pl.ds(rbase, _CHUNK)], out_sems[b])
        h_gi = pltpu.async_copy(gi_bufs[b], gio_hbm.at[pl.ds(rbase, _CHUNK)], out_sems[b])
        h_x = pltpu.async_copy(x_bufs[b], xui_hbm.at[pl.ds(rbase, _CHUNK)], out_sems[b])
        return (h_gu, h_gi, h_x)

    h_in = [None] * _RING
    h_out = [None] * _RING
    for ci in range(_PF):
        h_in[ci % _RING] = start_in(ci, ci % _RING)

    for ci in range(_NCHUNK):
        b = ci % _RING
        for h in h_in[b]:
            h.wait()
        _dot_chunk(gu_bufs[b], gi_bufs[b], x_bufs[b])
        h_out[b] = start_out(ci, b)
        pf = ci + _PF
        if pf < _NCHUNK:
            pb = pf % _RING
            if h_out[pb] is not None:
                for h in h_out[pb]:
                    h.wait()
                h_out[pb] = None
            h_in[pb] = start_in(pf, pb)

    for b in range(_RING):
        if h_out[b] is not None:
            for h in h_out[b]:
                h.wait()


def kernel(gu, gi):
    mesh = plsc.VectorSubcoreMesh(core_axis_name="c", subcore_axis_name="s")
    scratch = (
        [pltpu.VMEM((_CHUNK, _DIM), jnp.float32) for _ in range(2 * _RING)]
        + [pltpu.VMEM((_CHUNK,), jnp.float32) for _ in range(_RING)]
        + [pltpu.SemaphoreType.DMA for _ in range(2 * _RING)]
    )
    k = functools.partial(
        pl.kernel,
        mesh=mesh,
        out_type=[
            jax.ShapeDtypeStruct((_BATCH,), jnp.float32),
            jax.ShapeDtypeStruct((_BATCH, _DIM), jnp.float32),
            jax.ShapeDtypeStruct((_BATCH, _DIM), jnp.float32),
        ],
        scratch_types=scratch,
        compiler_params=pltpu.CompilerParams(needs_layout_passes=False),
    )(_sc_body)
    xui, guo, gio = k(gu, gi)
    return (xui, guo, gio)


# speedup vs baseline: 2.3254x; 2.3254x over previous
import jax
import jax.numpy as jnp
from jax.experimental import pallas as pl

_BATCH = 16384
_DIM = 256
_BLOCK = 4096


def _rowdot_copy_kernel(gu_ref, gi_ref, xui_ref, gu_out_ref, gi_out_ref):
    gu = gu_ref[:]
    gi = gi_ref[:]
    xui_ref[:] = jnp.sum(gu * gi, axis=1)
    gu_out_ref[:] = gu
    gi_out_ref[:] = gi


def kernel(gu, gi):
    xui, gu_out, gi_out = pl.pallas_call(
        _rowdot_copy_kernel,
        grid=(_BATCH // _BLOCK,),
        in_specs=[
            pl.BlockSpec((_BLOCK, _DIM), lambda i: (i, 0)),
            pl.BlockSpec((_BLOCK, _DIM), lambda i: (i, 0)),
        ],
        out_specs=[
            pl.BlockSpec((_BLOCK,), lambda i: (i,)),
            pl.BlockSpec((_BLOCK, _DIM), lambda i: (i, 0)),
            pl.BlockSpec((_BLOCK, _DIM), lambda i: (i, 0)),
        ],
        out_shape=[
            jax.ShapeDtypeStruct((_BATCH,), jnp.float32),
            jax.ShapeDtypeStruct((_BATCH, _DIM), jnp.float32),
            jax.ShapeDtypeStruct((_BATCH, _DIM), jnp.float32),
        ],
    )(gu, gi)
    return (xui, gu_out, gi_out)


# speedup vs baseline: 2.5386x; 1.0917x over previous
import jax
import jax.numpy as jnp
from jax.experimental import pallas as pl
from jax.experimental.pallas import tpu as pltpu

_BATCH = 16384
_DIM = 256
_SIZES = (1024, 1024, 2048, 4096, 4096, 2048, 1024, 1024)
_CHUNKS = []
_off = 0
for _sz in _SIZES:
    _CHUNKS.append((_off, _sz))
    _off += _sz
assert _off == _BATCH
_NCH = len(_CHUNKS)


def _body(gu_hbm, gi_hbm, xui_hbm, guo_hbm, gio_hbm,
          gu_v, gi_v, xui_v, insem, outsem, xsem):
    ins = []
    for c, (off, sz) in enumerate(_CHUNKS):
        a = pltpu.make_async_copy(
            gu_hbm.at[pl.ds(off, sz)], gu_v.at[pl.ds(off, sz)], insem.at[c])
        a.start()
        b = pltpu.make_async_copy(
            gi_hbm.at[pl.ds(off, sz)], gi_v.at[pl.ds(off, sz)], insem.at[c])
        b.start()
        ins.append((a, b))

    outs = []
    for c, (off, sz) in enumerate(_CHUNKS):
        a, b = ins[c]
        a.wait()
        b.wait()
        oa = pltpu.make_async_copy(
            gu_v.at[pl.ds(off, sz)], guo_hbm.at[pl.ds(off, sz)], outsem.at[c])
        oa.start()
        ob = pltpu.make_async_copy(
            gi_v.at[pl.ds(off, sz)], gio_hbm.at[pl.ds(off, sz)], outsem.at[c])
        ob.start()
        xui_v[pl.ds(off, sz)] = jnp.sum(
            gu_v[pl.ds(off, sz), :] * gi_v[pl.ds(off, sz), :], axis=1)
        ox = pltpu.make_async_copy(
            xui_v.at[pl.ds(off, sz)], xui_hbm.at[pl.ds(off, sz)], xsem.at[c])
        ox.start()
        outs += [oa, ob, ox]

    for h in outs:
        h.wait()


def kernel(gu, gi):
    xui, gu_out, gi_out = pl.pallas_call(
        _body,
        in_specs=[
            pl.BlockSpec(memory_space=pl.ANY),
            pl.BlockSpec(memory_space=pl.ANY),
        ],
        out_specs=[
            pl.BlockSpec(memory_space=pl.ANY),
            pl.BlockSpec(memory_space=pl.ANY),
            pl.BlockSpec(memory_space=pl.ANY),
        ],
        out_shape=[
            jax.ShapeDtypeStruct((_BATCH,), jnp.float32),
            jax.ShapeDtypeStruct((_BATCH, _DIM), jnp.float32),
            jax.ShapeDtypeStruct((_BATCH, _DIM), jnp.float32),
        ],
        scratch_shapes=[
            pltpu.VMEM((_BATCH, _DIM), jnp.float32),
            pltpu.VMEM((_BATCH, _DIM), jnp.float32),
            pltpu.VMEM((_BATCH,), jnp.float32),
            pltpu.SemaphoreType.DMA((_NCH,)),
            pltpu.SemaphoreType.DMA((_NCH,)),
            pltpu.SemaphoreType.DMA((_NCH,)),
        ],
    )(gu, gi)
    return (xui, gu_out, gi_out)
